# baseline (speedup 1.0000x reference)
import jax
import jax.numpy as jnp
from jax.experimental import pallas as pl


def _copy_block(x_ref, o_ref):
    o_ref[...] = x_ref[...]


def kernel(embedding_weight):
    n, d = embedding_weight.shape
    block_rows = 2000
    return pl.pallas_call(
        _copy_block,
        out_shape=jax.ShapeDtypeStruct((n, d), embedding_weight.dtype),
        grid=(n // block_rows,),
        in_specs=[pl.BlockSpec((block_rows, d), lambda i: (i, 0))],
        out_specs=pl.BlockSpec((block_rows, d), lambda i: (i, 0)),
    )(embedding_weight)
